# baseline (speedup 1.0000x reference)
import functools

import numpy as np

import jax
import jax.numpy as jnp
from jax import lax
from jax.experimental import pallas as pl
from jax.experimental.pallas import tpu as pltpu
from jax.experimental.pallas import tpu_sc as plsc

N = 10000
E = 320000
D = 128
NUM_ITERS = 4
GAMMA = 0.1
EPS = 0.1

NC = 2
NS = 16
NW = NC * NS

C = 64
NBUF = 4
CHP = 32
PH = 5
CH = CHP * PH
EPT = CH * C
E_PAD = EPT * NW

RPT = 632
N_PAD = RPT * NS

_mesh = plsc.VectorSubcoreMesh(
    core_axis_name="c", subcore_axis_name="s", num_cores=NC, num_subcores=NS
)


@functools.partial(
    pl.kernel,
    out_type=jax.ShapeDtypeStruct((NC, N_PAD, D), jnp.float32),
    mesh=_mesh,
    scratch_types=[
        pltpu.VMEM((2 * CHP, C), jnp.int32),
        pltpu.VMEM((2 * CHP, C), jnp.int32),
        [pltpu.VMEM((C, D), jnp.float32) for _ in range(NBUF)],
        pltpu.VMEM_SHARED((N_PAD, D), jnp.float32),
        [pltpu.SemaphoreType.DMA for _ in range(NBUF)],
        [pltpu.SemaphoreType.DMA for _ in range(NBUF)],
        [pltpu.SemaphoreType.DMA for _ in range(2)],
        [pltpu.SemaphoreType.DMA for _ in range(2)],
    ],
)
def _sc_propagate(h_hbm, src_hbm, dst_hbm, out_hbm, sidx, didx, rows, acc,
                  sems, scsems, ssems, dsems):
    cid = lax.axis_index("c")
    sid = lax.axis_index("s")
    wid = sid * NC + cid

    def _fire_idx(p):
        sl = (p % 2) * CHP
        pltpu.async_copy(src_hbm.at[wid, pl.ds(p * CHP, CHP)],
                         sidx.at[pl.ds(sl, CHP)], ssems[p % 2])
        pltpu.async_copy(dst_hbm.at[wid, pl.ds(p * CHP, CHP)],
                         didx.at[pl.ds(sl, CHP)], dsems[p % 2])

    def _wait_idx(p):
        sl = (p % 2) * CHP
        pltpu.make_async_copy(src_hbm.at[wid, pl.ds(p * CHP, CHP)],
                              sidx.at[pl.ds(sl, CHP)], ssems[p % 2]).wait()
        pltpu.make_async_copy(dst_hbm.at[wid, pl.ds(p * CHP, CHP)],
                              didx.at[pl.ds(sl, CHP)], dsems[p % 2]).wait()

    _fire_idx(0)
    _fire_idx(1)

    def _zero(i, _):
        rows[NBUF - 1][i // 8, pl.ds((i % 8) * 16, 16)] = jnp.zeros(
            (16,), jnp.float32)
        return 0

    lax.fori_loop(0, C * (D // 16), _zero, 0)

    _wait_idx(0)
    for b in range(NBUF - 1):
        pltpu.async_copy(h_hbm.at[sidx.at[b]], rows[b], sems[b])

    base = sid * RPT
    spans = [(i * C, min(C, RPT - i * C)) for i in range((RPT + C - 1) // C)]
    for off, sz in spans:
        pltpu.sync_copy(rows[NBUF - 1].at[pl.ds(0, sz)],
                        acc.at[pl.ds(base + off, sz)])
    pltpu.async_copy(h_hbm.at[sidx.at[NBUF - 1]], rows[NBUF - 1],
                     sems[NBUF - 1])
    plsc.subcore_barrier()

    for ph in range(PH):
        sb = (ph % 2) * CHP
        nb = ((ph + 1) % 2) * CHP
        if 1 <= ph < PH - 1:
            _fire_idx(ph + 1)

        def _body(j2, _):
            for b in range(NBUF):
                c = j2 * NBUF + b
                pltpu.make_async_copy(h_hbm.at[sidx.at[b]], rows[b],
                                      sems[b]).wait()
                pltpu.async_copy(rows[b], acc.at[didx.at[sb + c]], scsems[b],
                                 add=True)
            for b in range(NBUF):
                c = j2 * NBUF + b
                pltpu.make_async_copy(rows[b], acc.at[didx.at[sb + c]],
                                      scsems[b]).wait()
                pltpu.async_copy(h_hbm.at[sidx.at[sb + c + NBUF]], rows[b],
                                 sems[b])
            return 0

        lax.fori_loop(0, CHP // NBUF - 1, _body, 0)

        if ph + 1 < PH:
            _wait_idx(ph + 1)
        j2f = CHP // NBUF - 1
        for b in range(NBUF):
            c = j2f * NBUF + b
            pltpu.make_async_copy(h_hbm.at[sidx.at[b]], rows[b],
                                  sems[b]).wait()
            pltpu.async_copy(rows[b], acc.at[didx.at[sb + c]], scsems[b],
                             add=True)
        for b in range(NBUF):
            c = j2f * NBUF + b
            pltpu.make_async_copy(rows[b], acc.at[didx.at[sb + c]],
                                  scsems[b]).wait()
            if ph + 1 < PH:
                pltpu.async_copy(h_hbm.at[sidx.at[nb + b]], rows[b], sems[b])
    plsc.subcore_barrier()

    pltpu.sync_copy(acc.at[pl.ds(base, RPT)], out_hbm.at[cid, pl.ds(base, RPT)])


def _embed_body(x_ref, w_ref, b_ref, o_ref):
    z = lax.dot_general(
        x_ref[...], w_ref[...], (((1,), (1,)), ((), ())),
        preferred_element_type=jnp.float32,
    )
    o_ref[...] = jnp.maximum(z + b_ref[...], 0.0)


def _new_h(h_ref, p_ref, w_ref, lw_ref, b_ref):
    h = h_ref[...]
    p = p_ref[0] + p_ref[1]
    hwt = lax.dot_general(h, w_ref[...], (((1,), (1,)), ((), ())),
                          preferred_element_type=jnp.float32)
    hw = lax.dot_general(h, w_ref[...], (((1,), (0,)), ((), ())),
                         preferred_element_type=jnp.float32)
    plw = lax.dot_general(p, lw_ref[...], (((1,), (1,)), ((), ())),
                          preferred_element_type=jnp.float32)
    conv = hwt - hw - GAMMA * h + plw + b_ref[...]
    return h + EPS * jnp.tanh(conv)


def _update_body(h_ref, p_ref, w_ref, lw_ref, b_ref, o_ref):
    o_ref[...] = _new_h(h_ref, p_ref, w_ref, lw_ref, b_ref)


def _update_readout_body(h_ref, p_ref, w_ref, lw_ref, b_ref, rw_ref, rb_ref, o_ref):
    hn = _new_h(h_ref, p_ref, w_ref, lw_ref, b_ref)
    z = lax.dot_general(hn, rw_ref[...], (((1,), (1,)), ((), ())),
                        preferred_element_type=jnp.float32)
    o_ref[...] = z + rb_ref[...]


_ROWS_B = 2000
_GRID = (N // _ROWS_B,)
_row_spec = pl.BlockSpec((_ROWS_B, D), lambda i: (i, 0))
_parts_spec = pl.BlockSpec((NC, _ROWS_B, D), lambda i: (0, i, 0))
_mat_spec = pl.BlockSpec((D, D), lambda i: (0, 0))
_vec_spec = pl.BlockSpec((1, D), lambda i: (0, 0))
_out_struct = jax.ShapeDtypeStruct((N, D), jnp.float32)


def _tc_embed(x, w, b2):
    return pl.pallas_call(
        _embed_body, grid=_GRID,
        in_specs=[_row_spec, _mat_spec, _vec_spec],
        out_specs=_row_spec, out_shape=_out_struct,
    )(x, w, b2)


def _tc_update(h, parts, w, lw, b2):
    return pl.pallas_call(
        _update_body, grid=_GRID,
        in_specs=[_row_spec, _parts_spec, _mat_spec, _mat_spec, _vec_spec],
        out_specs=_row_spec, out_shape=_out_struct,
    )(h, parts, w, lw, b2)


def _tc_update_readout(h, parts, w, lw, b2, rw, rb2):
    return pl.pallas_call(
        _update_readout_body, grid=_GRID,
        in_specs=[_row_spec, _parts_spec, _mat_spec, _mat_spec, _vec_spec,
                  _mat_spec, _vec_spec],
        out_specs=_row_spec, out_shape=_out_struct,
    )(h, parts, w, lw, b2, rw, rb2)


def kernel(x, edge_index, emb_W, emb_b, W, bias, lin_W, readout_W, readout_b):
    ei = edge_index.astype(jnp.int32)
    pad = E_PAD - E
    pad_ar = np.arange(pad, dtype=np.int32)
    pad_src = jnp.asarray(pad_ar % N)
    pad_dst = jnp.asarray(N + (pad_ar % (N_PAD - N)))
    src = jnp.concatenate([ei[0], pad_src]).reshape(NW, CH, C)
    dst = jnp.concatenate([ei[1], pad_dst]).reshape(NW, CH, C)

    emb_b2 = emb_b.reshape(1, D)
    bias2 = bias.reshape(1, D)
    ro_b2 = readout_b.reshape(1, D)

    h = _tc_embed(x, emb_W, emb_b2)
    for _ in range(NUM_ITERS - 1):
        parts = _sc_propagate(h, src, dst)
        h = _tc_update(h, parts, W, lin_W, bias2)
    parts = _sc_propagate(h, src, dst)
    return _tc_update_readout(h, parts, W, lin_W, bias2, readout_W, ro_b2)


# speedup vs baseline: 1.1840x; 1.1840x over previous
import functools

import numpy as np

import jax
import jax.numpy as jnp
from jax import lax
from jax.experimental import pallas as pl
from jax.experimental.pallas import tpu as pltpu
from jax.experimental.pallas import tpu_sc as plsc

N = 10000
E = 320000
D = 128
NUM_ITERS = 4
GAMMA = 0.1
EPS = 0.1

NC = 2
NS = 16
NW = NC * NS

C = 64
NBUF = 4
CHP = 32
PH = 5
CH = CHP * PH
EPT = CH * C
E_PAD = EPT * NW

RPT = 632
N_PAD = RPT * NS

_mesh = plsc.VectorSubcoreMesh(
    core_axis_name="c", subcore_axis_name="s", num_cores=NC, num_subcores=NS
)


@functools.partial(
    pl.kernel,
    out_type=jax.ShapeDtypeStruct((NC, N_PAD, D), jnp.float32),
    mesh=_mesh,
    scratch_types=[
        pltpu.VMEM((2 * CHP, C), jnp.int32),
        pltpu.VMEM((2 * CHP, C), jnp.int32),
        [pltpu.VMEM((C, D), jnp.float32) for _ in range(NBUF)],
        pltpu.VMEM_SHARED((N_PAD, D), jnp.float32),
        [pltpu.SemaphoreType.DMA for _ in range(NBUF)],
        [pltpu.SemaphoreType.DMA for _ in range(2)],
        [pltpu.SemaphoreType.DMA for _ in range(2)],
    ],
)
def _sc_propagate(h_hbm, src_hbm, dst_hbm, out_hbm, sidx, didx, rows, acc,
                  sems, ssems, dsems):
    cid = lax.axis_index("c")
    sid = lax.axis_index("s")
    wid = sid * NC + cid

    def _fire_idx(p):
        sl = (p % 2) * CHP
        pltpu.async_copy(src_hbm.at[wid, pl.ds(p * CHP, CHP)],
                         sidx.at[pl.ds(sl, CHP)], ssems[p % 2])
        pltpu.async_copy(dst_hbm.at[wid, pl.ds(p * CHP, CHP)],
                         didx.at[pl.ds(sl, CHP)], dsems[p % 2])

    def _wait_idx(p):
        sl = (p % 2) * CHP
        pltpu.make_async_copy(src_hbm.at[wid, pl.ds(p * CHP, CHP)],
                              sidx.at[pl.ds(sl, CHP)], ssems[p % 2]).wait()
        pltpu.make_async_copy(dst_hbm.at[wid, pl.ds(p * CHP, CHP)],
                              didx.at[pl.ds(sl, CHP)], dsems[p % 2]).wait()

    _fire_idx(0)
    _fire_idx(1)

    def _zero(i, _):
        rows[NBUF - 1][i // 8, pl.ds((i % 8) * 16, 16)] = jnp.zeros(
            (16,), jnp.float32)
        return 0

    lax.fori_loop(0, C * (D // 16), _zero, 0)

    _wait_idx(0)
    for b in range(NBUF - 1):
        pltpu.async_copy(h_hbm.at[sidx.at[b]], rows[b], sems[b])

    base = sid * RPT
    spans = [(i * C, min(C, RPT - i * C)) for i in range((RPT + C - 1) // C)]
    for off, sz in spans:
        pltpu.sync_copy(rows[NBUF - 1].at[pl.ds(0, sz)],
                        acc.at[pl.ds(base + off, sz)])
    pltpu.async_copy(h_hbm.at[sidx.at[NBUF - 1]], rows[NBUF - 1],
                     sems[NBUF - 1])
    plsc.subcore_barrier()

    for ph in range(PH):
        sb = (ph % 2) * CHP
        nb = ((ph + 1) % 2) * CHP
        if 1 <= ph < PH - 1:
            _fire_idx(ph + 1)

        def _body(j2, _):
            for b in range(NBUF):
                c = j2 * NBUF + b
                pltpu.make_async_copy(h_hbm.at[sidx.at[b]], rows[b],
                                      sems[b]).wait()
                pltpu.sync_copy(rows[b], acc.at[didx.at[sb + c]], add=True)
                pltpu.async_copy(h_hbm.at[sidx.at[sb + c + NBUF]], rows[b],
                                 sems[b])
            return 0

        lax.fori_loop(0, CHP // NBUF - 1, _body, 0)

        if ph + 1 < PH:
            _wait_idx(ph + 1)
        j2f = CHP // NBUF - 1
        for b in range(NBUF):
            c = j2f * NBUF + b
            pltpu.make_async_copy(h_hbm.at[sidx.at[b]], rows[b],
                                  sems[b]).wait()
            pltpu.sync_copy(rows[b], acc.at[didx.at[sb + c]], add=True)
            if ph + 1 < PH:
                pltpu.async_copy(h_hbm.at[sidx.at[nb + b]], rows[b], sems[b])
    plsc.subcore_barrier()

    pltpu.sync_copy(acc.at[pl.ds(base, RPT)], out_hbm.at[cid, pl.ds(base, RPT)])


def _embed_body(x_ref, w_ref, b_ref, o_ref):
    z = lax.dot_general(
        x_ref[...], w_ref[...], (((1,), (1,)), ((), ())),
        preferred_element_type=jnp.float32,
    )
    o_ref[...] = jnp.maximum(z + b_ref[...], 0.0)


def _new_h(h_ref, p_ref, w_ref, lw_ref, b_ref):
    h = h_ref[...]
    p = p_ref[0] + p_ref[1]
    hwt = lax.dot_general(h, w_ref[...], (((1,), (1,)), ((), ())),
                          preferred_element_type=jnp.float32)
    hw = lax.dot_general(h, w_ref[...], (((1,), (0,)), ((), ())),
                         preferred_element_type=jnp.float32)
    plw = lax.dot_general(p, lw_ref[...], (((1,), (1,)), ((), ())),
                          preferred_element_type=jnp.float32)
    conv = hwt - hw - GAMMA * h + plw + b_ref[...]
    return h + EPS * jnp.tanh(conv)


def _update_body(h_ref, p_ref, w_ref, lw_ref, b_ref, o_ref):
    o_ref[...] = _new_h(h_ref, p_ref, w_ref, lw_ref, b_ref)


def _update_readout_body(h_ref, p_ref, w_ref, lw_ref, b_ref, rw_ref, rb_ref, o_ref):
    hn = _new_h(h_ref, p_ref, w_ref, lw_ref, b_ref)
    z = lax.dot_general(hn, rw_ref[...], (((1,), (1,)), ((), ())),
                        preferred_element_type=jnp.float32)
    o_ref[...] = z + rb_ref[...]


_ROWS_B = 2000
_GRID = (N // _ROWS_B,)
_row_spec = pl.BlockSpec((_ROWS_B, D), lambda i: (i, 0))
_parts_spec = pl.BlockSpec((NC, _ROWS_B, D), lambda i: (0, i, 0))
_mat_spec = pl.BlockSpec((D, D), lambda i: (0, 0))
_vec_spec = pl.BlockSpec((1, D), lambda i: (0, 0))
_out_struct = jax.ShapeDtypeStruct((N, D), jnp.float32)


def _tc_embed(x, w, b2):
    return pl.pallas_call(
        _embed_body, grid=_GRID,
        in_specs=[_row_spec, _mat_spec, _vec_spec],
        out_specs=_row_spec, out_shape=_out_struct,
    )(x, w, b2)


def _tc_update(h, parts, w, lw, b2):
    return pl.pallas_call(
        _update_body, grid=_GRID,
        in_specs=[_row_spec, _parts_spec, _mat_spec, _mat_spec, _vec_spec],
        out_specs=_row_spec, out_shape=_out_struct,
    )(h, parts, w, lw, b2)


def _tc_update_readout(h, parts, w, lw, b2, rw, rb2):
    return pl.pallas_call(
        _update_readout_body, grid=_GRID,
        in_specs=[_row_spec, _parts_spec, _mat_spec, _mat_spec, _vec_spec,
                  _mat_spec, _vec_spec],
        out_specs=_row_spec, out_shape=_out_struct,
    )(h, parts, w, lw, b2, rw, rb2)


def kernel(x, edge_index, emb_W, emb_b, W, bias, lin_W, readout_W, readout_b):
    ei = edge_index.astype(jnp.int32)
    pad = E_PAD - E
    pad_ar = np.arange(pad, dtype=np.int32)
    pad_src = jnp.asarray(pad_ar % N)
    pad_dst = jnp.asarray(N + (pad_ar % (N_PAD - N)))
    src = jnp.concatenate([ei[0], pad_src]).reshape(NW, CH, C)
    dst = jnp.concatenate([ei[1], pad_dst]).reshape(NW, CH, C)

    emb_b2 = emb_b.reshape(1, D)
    bias2 = bias.reshape(1, D)
    ro_b2 = readout_b.reshape(1, D)

    h = _tc_embed(x, emb_W, emb_b2)
    for _ in range(NUM_ITERS - 1):
        parts = _sc_propagate(h, src, dst)
        h = _tc_update(h, parts, W, lin_W, bias2)
    parts = _sc_propagate(h, src, dst)
    return _tc_update_readout(h, parts, W, lin_W, bias2, readout_W, ro_b2)
